# baseline (speedup 1.0000x reference)
import functools

import jax
import jax.numpy as jnp
from jax import lax
from jax.experimental import pallas as pl
from jax.experimental.pallas import tpu as pltpu
from jax.experimental.pallas import tpu_sc as plsc

VOCAB = 100000
DIM = 64
B = 4096
S = 200
NC = 2
NS = 16
NW = NC * NS
BLK = B // NW
LANES = 16
NVREG = DIM // LANES
TILE = 8 * BLK
EPS = 1e-5
_RSQRT_MAGIC = 0x5F3759DF

_GATHER_DNUMS = lax.GatherDimensionNumbers(
    offset_dims=(), collapsed_slice_dims=(0,), start_index_map=(0,))


def _xshuffle(v, k):
    perm = lax.iota(jnp.int32, LANES) ^ k
    return lax.gather(v, perm[:, None], _GATHER_DNUMS, (1,),
                      mode=lax.GatherScatterMode.PROMISE_IN_BOUNDS)


def _rsqrt(t):
    i = lax.bitcast_convert_type(t, jnp.int32)
    y = lax.bitcast_convert_type(_RSQRT_MAGIC - (i >> 1), jnp.float32)
    for _ in range(2):
        y = y * (1.5 - 0.5 * t * y * y)
    return y


def _pos_encoding():
    pos = jnp.arange(S, dtype=jnp.float32)[:, None]
    d = 2.0 * jnp.arange(DIM, dtype=jnp.float32) / DIM
    pe = pos / jnp.power(10000.0, d)
    pe = pe.at[:, 0::2].set(jnp.sin(pe[:, 0::2]))
    pe = pe.at[:, 1::2].set(jnp.cos(pe[:, 1::2]))
    return pe


def _sc_embed(xt, tok, pe, seg2, ln_scale, ln_bias):
    mesh = plsc.VectorSubcoreMesh(core_axis_name="c", subcore_axis_name="s")

    @functools.partial(
        pl.kernel,
        mesh=mesh,
        out_type=jax.ShapeDtypeStruct((S, DIM // 8, NW, 8, BLK), jnp.float32),
        scratch_types=[
            pltpu.VMEM((2, BLK), jnp.int32),
            pltpu.VMEM((2, BLK, DIM), jnp.float32),
            pltpu.VMEM((DIM // 8, 1, 8, BLK), jnp.float32),
            pltpu.VMEM((DIM // 8, 1, 8, BLK), jnp.float32),
            pltpu.VMEM((S, DIM), jnp.float32),
            pltpu.VMEM((S, DIM), jnp.float32),
            pltpu.VMEM((2, DIM), jnp.float32),
            pltpu.VMEM((DIM,), jnp.float32),
            pltpu.VMEM((DIM,), jnp.float32),
            pltpu.SemaphoreType.DMA,
            pltpu.SemaphoreType.DMA,
            pltpu.SemaphoreType.DMA,
            pltpu.SemaphoreType.DMA,
            pltpu.SemaphoreType.DMA,
            pltpu.SemaphoreType.DMA,
        ],
        compiler_params=pltpu.CompilerParams(use_tc_tiling_on_sc=False, needs_layout_passes=False),
    )
    def k(x_hbm, tok_hbm, pe_hbm, seg_hbm, gam_hbm, bet_hbm, out_hbm,
          idx_v, rows_v, ob0, ob1, add_v, pe_v, seg_v, gam_v, bet_v,
          gs0, gs1, is0, is1, os0, os1):
        gsem = (gs0, gs1)
        isem = (is0, is1)
        osem = (os0, os1)
        obuf = (ob0, ob1)
        wid = lax.axis_index("s") * NC + lax.axis_index("c")
        bbase = wid * BLK
        pltpu.sync_copy(pe_hbm, pe_v)
        pltpu.sync_copy(seg_hbm, seg_v)
        pltpu.sync_copy(gam_hbm, gam_v)
        pltpu.sync_copy(bet_hbm, bet_v)

        def build(i, c):
            srow = jnp.where(i >= S // 2 + 1, 1, 0)
            for g in range(NVREG):
                ds = pl.ds(g * LANES, LANES)
                add_v[i, ds] = pe_v[i, ds] + seg_v[srow, ds]
            return c
        lax.fori_loop(0, S, build, 0)

        def start_gather(buf):
            pltpu.async_copy(tok_hbm.at[idx_v.at[buf]], rows_v.at[buf],
                             gsem[buf])

        def wait_gather(buf):
            pltpu.make_async_copy(tok_hbm.at[idx_v.at[buf]], rows_v.at[buf],
                                  gsem[buf]).wait()

        _dt = [lax.iota(jnp.int32, LANES) // 8 + 2 * g for g in range(NVREG)]
        _dr = lax.iota(jnp.int32, LANES) % 8
        _z = lax.iota(jnp.int32, LANES) * 0

        def compute_pos(buf, pos, carry):
            a = [add_v[pos, pl.ds(g * LANES, LANES)] for g in range(NVREG)]

            @plsc.parallel_loop(0, BLK, step=1, unroll=4, carry=carry)
            def row(r, c):
                v = [rows_v[buf, r, pl.ds(g * LANES, LANES)] + a[g]
                     for g in range(NVREG)]
                sm = (v[0] + v[1]) + (v[2] + v[3])
                sq = ((v[0] * v[0] + v[1] * v[1])
                      + (v[2] * v[2] + v[3] * v[3]))
                for kk in (1, 2, 4, 8):
                    sm = sm + _xshuffle(sm, kk)
                    sq = sq + _xshuffle(sq, kk)
                mean = sm * (1.0 / DIM)
                var = sq * (1.0 / DIM) - mean * mean
                y = _rsqrt(var + EPS)
                col = jnp.full((LANES,), r, jnp.int32)
                for g in range(NVREG):
                    plsc.store_scatter(
                        obuf[buf], [_dt[g], _z, _dr, col],
                        (v[g] - mean) * (y * c[g]) + c[NVREG + g])
                return c
            return row

        pltpu.sync_copy(x_hbm.at[0, pl.ds(bbase, BLK)], idx_v.at[0])
        start_gather(0)
        pltpu.async_copy(x_hbm.at[1, pl.ds(bbase, BLK)], idx_v.at[1], isem[1])

        carry0 = (tuple(gam_v[pl.ds(g * LANES, LANES)] for g in range(NVREG))
                  + tuple(bet_v[pl.ds(g * LANES, LANES)] for g in range(NVREG)))

        def pair(p, carry):
            for cur in range(2):
                nxt = 1 - cur
                s = 2 * p + cur
                wait_gather(cur)

                @pl.when(s + 1 < S)
                def _():
                    pltpu.make_async_copy(x_hbm.at[s + 1, pl.ds(bbase, BLK)],
                                          idx_v.at[nxt], isem[nxt]).wait()
                    start_gather(nxt)

                @pl.when(s + 2 < S)
                def _():
                    pltpu.async_copy(x_hbm.at[s + 2, pl.ds(bbase, BLK)],
                                     idx_v.at[cur], isem[cur])

                @pl.when(s >= 2)
                def _():
                    pltpu.make_async_copy(
                        obuf[cur],
                        out_hbm.at[s - 2, :, pl.ds(wid, 1)],
                        osem[cur]).wait()

                carry = compute_pos(cur, s, carry)
                pltpu.async_copy(obuf[cur],
                                 out_hbm.at[s, :, pl.ds(wid, 1)],
                                 osem[cur])
            return carry

        lax.fori_loop(0, S // 2, pair, carry0)
        for cur in range(2):
            pltpu.make_async_copy(obuf[cur],
                                  out_hbm.at[cur, :, pl.ds(wid, 1)],
                                  osem[cur]).wait()

    return k(xt, tok, pe, seg2, ln_scale, ln_bias)


def kernel(x, token_table, segment_table, ln_scale, ln_bias):
    pe = _pos_encoding()
    seg2 = lax.slice_in_dim(segment_table, 0, 2)
    xt = x.T
    out5 = _sc_embed(xt, token_table, pe, seg2, ln_scale, ln_bias)
    return out5.transpose(2, 4, 0, 1, 3).reshape(B, S, DIM)


# speedup vs baseline: 1.0784x; 1.0784x over previous
import functools

import jax
import jax.numpy as jnp
from jax import lax
from jax.experimental import pallas as pl
from jax.experimental.pallas import tpu as pltpu
from jax.experimental.pallas import tpu_sc as plsc

VOCAB = 100000
DIM = 64
B = 4096
S = 200
NC = 2
NS = 16
NW = NC * NS
BLK = B // NW
LANES = 16
NVREG = DIM // LANES
TILE = 8 * BLK
EPS = 1e-5
_RSQRT_MAGIC = 0x5F3759DF

_GATHER_DNUMS = lax.GatherDimensionNumbers(
    offset_dims=(), collapsed_slice_dims=(0,), start_index_map=(0,))


def _xshuffle(v, k):
    perm = lax.iota(jnp.int32, LANES) ^ k
    return lax.gather(v, perm[:, None], _GATHER_DNUMS, (1,),
                      mode=lax.GatherScatterMode.PROMISE_IN_BOUNDS)


def _rsqrt(t):
    i = lax.bitcast_convert_type(t, jnp.int32)
    y = lax.bitcast_convert_type(_RSQRT_MAGIC - (i >> 1), jnp.float32)
    for _ in range(2):
        y = y * (1.5 - 0.5 * t * y * y)
    return y


def _pos_encoding():
    pos = jnp.arange(S, dtype=jnp.float32)[:, None]
    d = 2.0 * jnp.arange(DIM, dtype=jnp.float32) / DIM
    pe = pos / jnp.power(10000.0, d)
    pe = pe.at[:, 0::2].set(jnp.sin(pe[:, 0::2]))
    pe = pe.at[:, 1::2].set(jnp.cos(pe[:, 1::2]))
    return pe


def _sc_embed(xt, tok, pe, seg2, ln_scale, ln_bias):
    mesh = plsc.VectorSubcoreMesh(core_axis_name="c", subcore_axis_name="s")

    @functools.partial(
        pl.kernel,
        mesh=mesh,
        out_type=jax.ShapeDtypeStruct((S, DIM // 8, NW, 8, BLK), jnp.float32),
        scratch_types=[
            pltpu.VMEM((2, BLK), jnp.int32),
            pltpu.VMEM((2, BLK, DIM), jnp.float32),
            pltpu.VMEM((DIM // 8, 1, 8, BLK), jnp.float32),
            pltpu.VMEM((DIM // 8, 1, 8, BLK), jnp.float32),
            pltpu.VMEM((S, DIM), jnp.float32),
            pltpu.VMEM((S, DIM), jnp.float32),
            pltpu.VMEM((2, DIM), jnp.float32),
            pltpu.VMEM((DIM,), jnp.float32),
            pltpu.VMEM((DIM,), jnp.float32),
            pltpu.SemaphoreType.DMA,
            pltpu.SemaphoreType.DMA,
            pltpu.SemaphoreType.DMA,
            pltpu.SemaphoreType.DMA,
            pltpu.SemaphoreType.DMA,
            pltpu.SemaphoreType.DMA,
        ],
        compiler_params=pltpu.CompilerParams(use_tc_tiling_on_sc=False, needs_layout_passes=False),
    )
    def k(x_hbm, tok_hbm, pe_hbm, seg_hbm, gam_hbm, bet_hbm, out_hbm,
          idx_v, rows_v, ob0, ob1, add_v, pe_v, seg_v, gam_v, bet_v,
          gs0, gs1, is0, is1, os0, os1):
        gsem = (gs0, gs1)
        isem = (is0, is1)
        osem = (os0, os1)
        obuf = (ob0, ob1)
        wid = lax.axis_index("s") * NC + lax.axis_index("c")
        bbase = wid * BLK
        pltpu.sync_copy(pe_hbm, pe_v)
        pltpu.sync_copy(seg_hbm, seg_v)
        pltpu.sync_copy(gam_hbm, gam_v)
        pltpu.sync_copy(bet_hbm, bet_v)

        def build(i, c):
            srow = jnp.where(i >= S // 2 + 1, 1, 0)
            for g in range(NVREG):
                ds = pl.ds(g * LANES, LANES)
                add_v[i, ds] = pe_v[i, ds] + seg_v[srow, ds]
            return c
        lax.fori_loop(0, S, build, 0)

        def start_gather(buf):
            pltpu.async_copy(tok_hbm.at[idx_v.at[buf]], rows_v.at[buf],
                             gsem[buf])

        def wait_gather(buf):
            pltpu.make_async_copy(tok_hbm.at[idx_v.at[buf]], rows_v.at[buf],
                                  gsem[buf]).wait()

        _lane = lax.iota(jnp.int32, LANES)

        def compute_pos(buf, pos, carry):
            a = [add_v[pos, pl.ds(g * LANES, LANES)] for g in range(NVREG)]

            @plsc.parallel_loop(0, BLK, step=1, unroll=4, carry=carry)
            def row(r, c):
                v = [rows_v[buf, r, pl.ds(g * LANES, LANES)] + a[g]
                     for g in range(NVREG)]
                sm = (v[0] + v[1]) + (v[2] + v[3])
                sq = ((v[0] * v[0] + v[1] * v[1])
                      + (v[2] * v[2] + v[3] * v[3]))
                for kk in (1, 2, 4, 8):
                    sm = sm + _xshuffle(sm, kk)
                    sq = sq + _xshuffle(sq, kk)
                mean = sm * (1.0 / DIM)
                var = sq * (1.0 / DIM) - mean * mean
                y = _rsqrt(var + EPS)
                for g in range(NVREG):
                    rows_v[buf, r, pl.ds(g * LANES, LANES)] = (
                        (v[g] - mean) * (y * c[g]) + c[NVREG + g])
                return c

            @plsc.parallel_loop(0, DIM, step=1, unroll=2, carry=row)
            def tr(d, c):
                dcol = jnp.full((LANES,), d, jnp.int32)
                for bc in range(BLK // LANES):
                    vec = plsc.load_gather(
                        rows_v.at[buf], [_lane + bc * LANES, dcol])
                    obuf[buf][d // 8, 0, d % 8, pl.ds(bc * LANES, LANES)] = vec
                return c
            return tr

        pltpu.sync_copy(x_hbm.at[0, pl.ds(bbase, BLK)], idx_v.at[0])
        start_gather(0)
        pltpu.async_copy(x_hbm.at[1, pl.ds(bbase, BLK)], idx_v.at[1], isem[1])

        carry0 = (tuple(gam_v[pl.ds(g * LANES, LANES)] for g in range(NVREG))
                  + tuple(bet_v[pl.ds(g * LANES, LANES)] for g in range(NVREG)))

        def pair(p, carry):
            for cur in range(2):
                nxt = 1 - cur
                s = 2 * p + cur
                wait_gather(cur)

                @pl.when(s + 1 < S)
                def _():
                    pltpu.make_async_copy(x_hbm.at[s + 1, pl.ds(bbase, BLK)],
                                          idx_v.at[nxt], isem[nxt]).wait()
                    start_gather(nxt)

                @pl.when(s + 2 < S)
                def _():
                    pltpu.async_copy(x_hbm.at[s + 2, pl.ds(bbase, BLK)],
                                     idx_v.at[cur], isem[cur])

                @pl.when(s >= 2)
                def _():
                    pltpu.make_async_copy(
                        obuf[cur],
                        out_hbm.at[s - 2, :, pl.ds(wid, 1)],
                        osem[cur]).wait()

                carry = compute_pos(cur, s, carry)
                pltpu.async_copy(obuf[cur],
                                 out_hbm.at[s, :, pl.ds(wid, 1)],
                                 osem[cur])
            return carry

        lax.fori_loop(0, S // 2, pair, carry0)
        for cur in range(2):
            pltpu.make_async_copy(obuf[cur],
                                  out_hbm.at[cur, :, pl.ds(wid, 1)],
                                  osem[cur]).wait()

    return k(xt, tok, pe, seg2, ln_scale, ln_bias)


def kernel(x, token_table, segment_table, ln_scale, ln_bias):
    pe = _pos_encoding()
    seg2 = lax.slice_in_dim(segment_table, 0, 2)
    xt = x.T
    out5 = _sc_embed(xt, token_table, pe, seg2, ln_scale, ln_bias)
    return out5.transpose(2, 4, 0, 1, 3).reshape(B, S, DIM)
